# baseline (speedup 1.0000x reference)
import jax
import jax.numpy as jnp
from jax.experimental import pallas as pl
from jax.experimental.pallas import tpu as pltpu

_N_EXPERTS = 8
_D_MODEL = 768
_N_TOKENS = 2048


def _moe_kernel(x_ref, wg_ref, we_ref, be_ref, out_ref, gw_ref, xb_ref,
                acc_ref):
    e = pl.program_id(0)

    @pl.when(e == 0)
    def _prologue():
        x = x_ref[...]
        logits = jax.lax.dot_general(
            x, wg_ref[...], (((1,), (1,)), ((), ())),
            preferred_element_type=jnp.float32)
        g = jax.nn.softmax(logits, axis=1)
        e_iota = jax.lax.broadcasted_iota(jnp.int32, (_N_TOKENS, _N_EXPERTS), 1)
        m1 = jnp.max(g, axis=1, keepdims=True)
        i1 = jnp.min(jnp.where(g == m1, e_iota, _N_EXPERTS), axis=1,
                     keepdims=True)
        g2 = jnp.where(e_iota == i1, -jnp.inf, g)
        m2 = jnp.max(g2, axis=1, keepdims=True)
        i2 = jnp.min(jnp.where(g2 == m2, e_iota, _N_EXPERTS), axis=1,
                     keepdims=True)
        gw = jnp.where((e_iota == i1) | (e_iota == i2), g, 0.0)
        gw_ref[...] = gw
        xb_ref[...] = x.astype(jnp.bfloat16)
        acc_ref[...] = jax.lax.dot_general(
            gw, be_ref[...], (((1,), (0,)), ((), ())),
            precision=jax.lax.Precision.HIGHEST,
            preferred_element_type=jnp.float32)

    ye = jax.lax.dot_general(
        xb_ref[...], we_ref[0].astype(jnp.bfloat16), (((1,), (1,)), ((), ())),
        preferred_element_type=jnp.float32)
    col = jax.lax.broadcasted_iota(jnp.int32, (_N_TOKENS, _N_EXPERTS), 1)
    wcol = jnp.sum(jnp.where(col == e, gw_ref[...], 0.0), axis=1,
                   keepdims=True)

    @pl.when(e < _N_EXPERTS - 1)
    def _accum():
        acc_ref[...] += wcol * ye

    @pl.when(e == _N_EXPERTS - 1)
    def _final():
        out_ref[...] = acc_ref[...] + wcol * ye


def kernel(input_data, W_gate, W_experts, b_experts):
    return pl.pallas_call(
        _moe_kernel,
        grid=(_N_EXPERTS,),
        in_specs=[
            pl.BlockSpec((_N_TOKENS, _D_MODEL), lambda e: (0, 0)),
            pl.BlockSpec((_N_EXPERTS, _D_MODEL), lambda e: (0, 0)),
            pl.BlockSpec((1, _D_MODEL, _D_MODEL), lambda e: (e, 0, 0)),
            pl.BlockSpec((_N_EXPERTS, _D_MODEL), lambda e: (0, 0)),
        ],
        out_specs=pl.BlockSpec((_N_TOKENS, _D_MODEL), lambda e: (0, 0)),
        out_shape=jax.ShapeDtypeStruct((_N_TOKENS, _D_MODEL), jnp.float32),
        scratch_shapes=[
            pltpu.VMEM((_N_TOKENS, _N_EXPERTS), jnp.float32),
            pltpu.VMEM((_N_TOKENS, _D_MODEL), jnp.bfloat16),
            pltpu.VMEM((_N_TOKENS, _D_MODEL), jnp.float32),
        ],
    )(input_data, W_gate, W_experts, b_experts)


# speedup vs baseline: 1.0041x; 1.0041x over previous
import jax
import jax.numpy as jnp
from jax.experimental import pallas as pl
from jax.experimental.pallas import tpu as pltpu

_N_EXPERTS = 8
_D_MODEL = 768
_N_TOKENS = 2048


def _moe_kernel(x_ref, wg_ref, we_hbm, be_ref, out_ref, wbuf, sem):
    def start_copy(e):
        pltpu.make_async_copy(
            we_hbm.at[e], wbuf.at[e % 2], sem.at[e % 2]).start()

    start_copy(0)
    start_copy(1)

    x = x_ref[...]
    logits = jax.lax.dot_general(
        x, wg_ref[...], (((1,), (1,)), ((), ())),
        preferred_element_type=jnp.float32)
    g = jax.nn.softmax(logits, axis=1)
    e_iota = jax.lax.broadcasted_iota(jnp.int32, (_N_TOKENS, _N_EXPERTS), 1)
    m1 = jnp.max(g, axis=1, keepdims=True)
    i1 = jnp.min(jnp.where(g == m1, e_iota, _N_EXPERTS), axis=1, keepdims=True)
    g2 = jnp.where(e_iota == i1, -jnp.inf, g)
    m2 = jnp.max(g2, axis=1, keepdims=True)
    i2 = jnp.min(jnp.where(g2 == m2, e_iota, _N_EXPERTS), axis=1, keepdims=True)
    gw = jnp.where((e_iota == i1) | (e_iota == i2), g, 0.0)

    xb = x.astype(jnp.bfloat16)
    acc = jax.lax.dot_general(
        gw, be_ref[...], (((1,), (0,)), ((), ())),
        precision=jax.lax.Precision.HIGHEST,
        preferred_element_type=jnp.float32)

    for e in range(_N_EXPERTS):
        pltpu.make_async_copy(
            we_hbm.at[e], wbuf.at[e % 2], sem.at[e % 2]).wait()
        ye = jax.lax.dot_general(
            xb, wbuf[e % 2].astype(jnp.bfloat16), (((1,), (1,)), ((), ())),
            preferred_element_type=jnp.float32)
        if e + 2 < _N_EXPERTS:
            start_copy(e + 2)
        acc = acc + gw[:, e:e + 1] * ye
    out_ref[...] = acc


def kernel(input_data, W_gate, W_experts, b_experts):
    return pl.pallas_call(
        _moe_kernel,
        in_specs=[
            pl.BlockSpec((_N_TOKENS, _D_MODEL), lambda: (0, 0)),
            pl.BlockSpec((_N_EXPERTS, _D_MODEL), lambda: (0, 0)),
            pl.BlockSpec(memory_space=pltpu.MemorySpace.HBM),
            pl.BlockSpec((_N_EXPERTS, _D_MODEL), lambda: (0, 0)),
        ],
        out_specs=pl.BlockSpec((_N_TOKENS, _D_MODEL), lambda: (0, 0)),
        out_shape=jax.ShapeDtypeStruct((_N_TOKENS, _D_MODEL), jnp.float32),
        scratch_shapes=[
            pltpu.VMEM((2, _D_MODEL, _D_MODEL), jnp.float32),
            pltpu.SemaphoreType.DMA((2,)),
        ],
    )(input_data, W_gate, W_experts, b_experts)


# speedup vs baseline: 1.1996x; 1.1948x over previous
import jax
import jax.numpy as jnp
from jax.experimental import pallas as pl
from jax.experimental.pallas import tpu as pltpu

_N_EXPERTS = 8
_D_MODEL = 768
_N_TOKENS = 2048


def _moe_kernel(x_ref, wg_ref, we_hbm, be_ref, out_ref, wbuf, sem):
    def start_copy(c):
        pltpu.make_async_copy(
            we_hbm.at[pl.ds(4 * c, 4)], wbuf.at[c], sem.at[c]).start()

    start_copy(0)
    start_copy(1)

    x = x_ref[...]
    logits = jax.lax.dot_general(
        x, wg_ref[...], (((1,), (1,)), ((), ())),
        preferred_element_type=jnp.float32)
    g = jax.nn.softmax(logits, axis=1)
    e_iota = jax.lax.broadcasted_iota(jnp.int32, (_N_TOKENS, _N_EXPERTS), 1)
    m1 = jnp.max(g, axis=1, keepdims=True)
    i1 = jnp.min(jnp.where(g == m1, e_iota, _N_EXPERTS), axis=1, keepdims=True)
    g2 = jnp.where(e_iota == i1, -jnp.inf, g)
    m2 = jnp.max(g2, axis=1, keepdims=True)
    i2 = jnp.min(jnp.where(g2 == m2, e_iota, _N_EXPERTS), axis=1, keepdims=True)
    gw = jnp.where((e_iota == i1) | (e_iota == i2), g, 0.0)

    xb = x.astype(jnp.bfloat16)
    acc = jax.lax.dot_general(
        gw, be_ref[...], (((1,), (0,)), ((), ())),
        precision=jax.lax.Precision.HIGHEST,
        preferred_element_type=jnp.float32)

    for c in range(2):
        pltpu.make_async_copy(
            we_hbm.at[pl.ds(4 * c, 4)], wbuf.at[c], sem.at[c]).wait()
        for j in range(4):
            e = 4 * c + j
            ye = jax.lax.dot_general(
                xb, wbuf[c, j].astype(jnp.bfloat16), (((1,), (1,)), ((), ())),
                preferred_element_type=jnp.float32)
            acc = acc + gw[:, e:e + 1] * ye
    out_ref[...] = acc


def kernel(input_data, W_gate, W_experts, b_experts):
    return pl.pallas_call(
        _moe_kernel,
        in_specs=[
            pl.BlockSpec((_N_TOKENS, _D_MODEL), lambda: (0, 0)),
            pl.BlockSpec((_N_EXPERTS, _D_MODEL), lambda: (0, 0)),
            pl.BlockSpec(memory_space=pltpu.MemorySpace.HBM),
            pl.BlockSpec((_N_EXPERTS, _D_MODEL), lambda: (0, 0)),
        ],
        out_specs=pl.BlockSpec((_N_TOKENS, _D_MODEL), lambda: (0, 0)),
        out_shape=jax.ShapeDtypeStruct((_N_TOKENS, _D_MODEL), jnp.float32),
        scratch_shapes=[
            pltpu.VMEM((2, 4, _D_MODEL, _D_MODEL), jnp.float32),
            pltpu.SemaphoreType.DMA((2,)),
        ],
    )(input_data, W_gate, W_experts, b_experts)
